# Initial kernel scaffold; baseline (speedup 1.0000x reference)
import jax
import jax.numpy as jnp
from jax.experimental import pallas as pl


def kernel(x, edge_index, edge_attr, u, batch, params):
    raise NotImplementedError("write your pallas kernel here")


# speedup vs baseline: 2.8075x; 2.8075x over previous
import functools

import jax
import jax.numpy as jnp
from jax import lax
from jax.experimental import pallas as pl
from jax.experimental.pallas import tpu as pltpu
from jax.experimental.pallas import tpu_sc as plsc

N = 10000
NPAD = 10240
E = 320000
FX = 128
FE = 16
FO = 128
D1 = FX + FE
D2 = FX + FO
EPS = 1e-5

NC = 2
NS = 16
NW = NC * NS
CH = 128
NCHUNK = E // CH
ROWS_PER_TILE = NPAD // NS

ECHUNK = 2560
EGRID = E // ECHUNK


def _mm_nt(a, b):
    return lax.dot_general(a, b, (((1,), (1,)), ((), ())),
                           preferred_element_type=jnp.float32)


def _elu(a):
    return jnp.where(a > 0, a, jnp.exp(a) - 1.0)


def _sc_mesh():
    return plsc.VectorSubcoreMesh(core_axis_name="c", subcore_axis_name="s",
                                  num_cores=NC, num_subcores=NS)


def _wid():
    return lax.axis_index("s") * NC + lax.axis_index("c")


def _sc_counts(row, col, z128, ones128):
    @functools.partial(
        pl.kernel,
        out_type=jax.ShapeDtypeStruct((NC, NPAD, FO), jnp.float32),
        mesh=_sc_mesh(),
        scratch_types=[pltpu.VMEM((1, CH), jnp.int32),
                       pltpu.VMEM((CH, FO), jnp.float32),
                       pltpu.VMEM_SHARED((NPAD, FO), jnp.float32)],
    )
    def k(row_h, col_h, z_h, ones_h, out_h, idx_v, ones_v, acc):
        cid = lax.axis_index("c")
        sid = lax.axis_index("s")
        r0 = sid * ROWS_PER_TILE
        pltpu.sync_copy(ones_h, ones_v)
        pltpu.sync_copy(z_h.at[pl.ds(r0, ROWS_PER_TILE)],
                        acc.at[pl.ds(r0, ROWS_PER_TILE)])
        plsc.subcore_barrier()

        trips = (NCHUNK - sid + NS - 1) // NS

        def step(j, carry):
            base = (sid + j * NS) * CH

            @pl.when(cid == 0)
            def _():
                pltpu.sync_copy(row_h.at[pl.ds(base, CH)], idx_v.at[0])

            @pl.when(cid == 1)
            def _():
                pltpu.sync_copy(col_h.at[pl.ds(base, CH)], idx_v.at[0])

            pltpu.sync_copy(ones_v, acc.at[idx_v.at[0]], add=True)
            return carry

        lax.fori_loop(0, trips, step, 0)
        plsc.subcore_barrier()
        pltpu.sync_copy(acc.at[pl.ds(r0, ROWS_PER_TILE)],
                        out_h.at[cid, pl.ds(r0, ROWS_PER_TILE)])

    return k(row, col, z128, ones128)


def _sc_gather(ytab, row):
    @functools.partial(
        pl.kernel,
        out_type=jax.ShapeDtypeStruct((E, FO), jnp.float32),
        mesh=_sc_mesh(),
        scratch_types=[pltpu.VMEM((1, CH), jnp.int32),
                       pltpu.VMEM((CH, FO), jnp.float32),
                       pltpu.SemaphoreType.DMA],
    )
    def k(y_h, row_h, g_h, idx_v, rows_v, sem):
        w = _wid()
        trips = (NCHUNK - w + NW - 1) // NW

        def step(j, carry):
            base = (w + j * NW) * CH
            pltpu.sync_copy(row_h.at[pl.ds(base, CH)], idx_v.at[0])
            pltpu.async_copy(y_h.at[idx_v.at[0]], rows_v, sem).wait()
            pltpu.sync_copy(rows_v, g_h.at[pl.ds(base, CH)])
            return carry

        lax.fori_loop(0, trips, step, 0)

    return k(ytab, row)


def _sc_scatter(e2, col, z128):
    @functools.partial(
        pl.kernel,
        out_type=jax.ShapeDtypeStruct((NC, NPAD, FO), jnp.float32),
        mesh=_sc_mesh(),
        scratch_types=[pltpu.VMEM((1, CH), jnp.int32),
                       pltpu.VMEM((CH, FO), jnp.float32),
                       pltpu.VMEM_SHARED((NPAD, FO), jnp.float32)],
    )
    def k(e2_h, col_h, z_h, out_h, idx_v, val_v, acc):
        cid = lax.axis_index("c")
        sid = lax.axis_index("s")
        w = _wid()
        r0 = sid * ROWS_PER_TILE
        pltpu.sync_copy(z_h.at[pl.ds(r0, ROWS_PER_TILE)],
                        acc.at[pl.ds(r0, ROWS_PER_TILE)])
        plsc.subcore_barrier()

        trips = (NCHUNK - w + NW - 1) // NW

        def step(j, carry):
            base = (w + j * NW) * CH
            pltpu.sync_copy(col_h.at[pl.ds(base, CH)], idx_v.at[0])
            pltpu.sync_copy(e2_h.at[pl.ds(base, CH)], val_v)
            pltpu.sync_copy(val_v, acc.at[idx_v.at[0]], add=True)
            return carry

        lax.fori_loop(0, trips, step, 0)
        plsc.subcore_barrier()
        pltpu.sync_copy(acc.at[pl.ds(r0, ROWS_PER_TILE)],
                        out_h.at[cid, pl.ds(r0, ROWS_PER_TILE)])

    return k(e2, col, z128)


def _tc_ea_stats(ea):
    def body(ea_ref, out_ref, acc_ref):
        i = pl.program_id(0)

        @pl.when(i == 0)
        def _():
            acc_ref[...] = jnp.zeros_like(acc_ref)

        blk = ea_ref[...]
        acc_ref[0:1, :] += jnp.sum(blk, axis=0, keepdims=True)
        acc_ref[1:2, :] += jnp.sum(blk * blk, axis=0, keepdims=True)

        @pl.when(i == EGRID - 1)
        def _():
            out_ref[...] = acc_ref[...]

    return pl.pallas_call(
        body,
        grid=(EGRID,),
        in_specs=[pl.BlockSpec((ECHUNK, FE), lambda i: (i, 0))],
        out_specs=pl.BlockSpec((8, FE), lambda i: (0, 0)),
        out_shape=jax.ShapeDtypeStruct((8, FE), jnp.float32),
        scratch_shapes=[pltpu.VMEM((8, FE), jnp.float32)],
    )(ea)


def _tc_prep(x, cnt2, easum, W1, g1r, b1r, c1r):
    def body(x_ref, cnt_ref, eas_ref, w1_ref, g1_ref, b1_ref, c1_ref,
             y_ref, w1es_ref, b1row_ref):
        cnt = cnt_ref[0, :N, 0:1]
        xv = x_ref[...]
        einv = 1.0 / E
        sum_x = jnp.sum(xv * cnt, axis=0, keepdims=True)
        sumsq_x = jnp.sum(xv * xv * cnt, axis=0, keepdims=True)
        mu_x = sum_x * einv
        var_x = sumsq_x * einv - mu_x * mu_x
        sc_x = g1_ref[0:1, :FX] * lax.rsqrt(var_x + EPS)
        sh_x = b1_ref[0:1, :FX] - mu_x * sc_x

        mu_e = eas_ref[0:1, :] * einv
        var_e = eas_ref[1:2, :] * einv - mu_e * mu_e
        sc_e = g1_ref[0:1, FX:] * lax.rsqrt(var_e + EPS)
        sh_e = b1_ref[0:1, FX:] - mu_e * sc_e

        w1 = w1_ref[...]
        w1x = w1[:, :FX]
        w1e = w1[:, FX:]
        y_ref[...] = _mm_nt(xv, w1x * sc_x)
        w1es_ref[...] = w1e * sc_e
        b1row_ref[...] = _mm_nt(sh_x, w1x) + _mm_nt(sh_e, w1e) + c1_ref[...]

    return pl.pallas_call(
        body,
        out_shape=[jax.ShapeDtypeStruct((N, FO), jnp.float32),
                   jax.ShapeDtypeStruct((FO, FE), jnp.float32),
                   jax.ShapeDtypeStruct((1, FO), jnp.float32)],
    )(x, cnt2, easum, W1, g1r, b1r, c1r)


def _tc_pass_a(G, ea, w1es, b1row):
    def body(g_ref, ea_ref, w_ref, b_ref, out_ref, acc_ref):
        i = pl.program_id(0)

        @pl.when(i == 0)
        def _():
            acc_ref[...] = jnp.zeros_like(acc_ref)

        a = g_ref[...] + _mm_nt(ea_ref[...], w_ref[...]) + b_ref[...]
        e1 = _elu(a)
        acc_ref[0:1, :] += jnp.sum(e1, axis=0, keepdims=True)
        acc_ref[1:2, :] += jnp.sum(e1 * e1, axis=0, keepdims=True)

        @pl.when(i == EGRID - 1)
        def _():
            out_ref[...] = acc_ref[...]

    return pl.pallas_call(
        body,
        grid=(EGRID,),
        in_specs=[pl.BlockSpec((ECHUNK, FO), lambda i: (i, 0)),
                  pl.BlockSpec((ECHUNK, FE), lambda i: (i, 0)),
                  pl.BlockSpec((FO, FE), lambda i: (0, 0)),
                  pl.BlockSpec((1, FO), lambda i: (0, 0))],
        out_specs=pl.BlockSpec((8, FO), lambda i: (0, 0)),
        out_shape=jax.ShapeDtypeStruct((8, FO), jnp.float32),
        scratch_shapes=[pltpu.VMEM((8, FO), jnp.float32)],
    )(G, ea, w1es, b1row)


def _tc_pass_b(G, ea, w1es, b1row, w2s, b2row):
    def body(g_ref, ea_ref, w_ref, b_ref, w2_ref, b2_ref,
             e2_ref, out_ref, acc_ref):
        i = pl.program_id(0)

        @pl.when(i == 0)
        def _():
            acc_ref[...] = jnp.zeros_like(acc_ref)

        a = g_ref[...] + _mm_nt(ea_ref[...], w_ref[...]) + b_ref[...]
        e1 = _elu(a)
        e2 = _elu(_mm_nt(e1, w2_ref[...]) + b2_ref[...])
        e2_ref[...] = e2
        acc_ref[0:1, :] += jnp.sum(e2, axis=0, keepdims=True)
        acc_ref[1:2, :] += jnp.sum(e2 * e2, axis=0, keepdims=True)

        @pl.when(i == EGRID - 1)
        def _():
            out_ref[...] = acc_ref[...]

    return pl.pallas_call(
        body,
        grid=(EGRID,),
        in_specs=[pl.BlockSpec((ECHUNK, FO), lambda i: (i, 0)),
                  pl.BlockSpec((ECHUNK, FE), lambda i: (i, 0)),
                  pl.BlockSpec((FO, FE), lambda i: (0, 0)),
                  pl.BlockSpec((1, FO), lambda i: (0, 0)),
                  pl.BlockSpec((FO, FO), lambda i: (0, 0)),
                  pl.BlockSpec((1, FO), lambda i: (0, 0))],
        out_specs=[pl.BlockSpec((ECHUNK, FO), lambda i: (i, 0)),
                   pl.BlockSpec((8, FO), lambda i: (0, 0))],
        out_shape=[jax.ShapeDtypeStruct((E, FO), jnp.float32),
                   jax.ShapeDtypeStruct((8, FO), jnp.float32)],
        scratch_shapes=[pltpu.VMEM((8, FO), jnp.float32)],
    )(G, ea, w1es, b1row, w2s, b2row)


def _tc_final(parts, cnt2, x, w3s, d3row, p):
    def body(parts_ref, cnt_ref, x_ref, w3_ref, d3_ref,
             g1_ref, b1_ref, mw1_ref, c1_ref,
             g2_ref, b2_ref, mw2_ref, c2_ref,
             g3_ref, b3_ref, mw3_ref, c3_ref, out_ref):
        seg = parts_ref[0, :N, :] + parts_ref[1, :N, :]
        cnt = cnt_ref[1, :N, 0:1]
        sums = _mm_nt(seg, w3_ref[...]) + cnt * d3_ref[...]
        mean = sums / jnp.maximum(cnt, 1.0)
        h = jnp.concatenate([x_ref[...], mean], axis=1)

        def bn(v, g, b):
            mu = jnp.mean(v, axis=0, keepdims=True)
            var = jnp.mean(v * v, axis=0, keepdims=True) - mu * mu
            s = g * lax.rsqrt(var + EPS)
            return v * s + (b - mu * s)

        h = bn(h, g1_ref[...], b1_ref[...])
        h = _elu(_mm_nt(h, mw1_ref[...]) + c1_ref[...])
        h = bn(h, g2_ref[...], b2_ref[...])
        h = _elu(_mm_nt(h, mw2_ref[...]) + c2_ref[...])
        h = bn(h, g3_ref[...], b3_ref[...])
        out_ref[...] = _mm_nt(h, mw3_ref[...]) + c3_ref[...]

    r = lambda v: v.reshape(1, -1)
    return pl.pallas_call(
        body,
        out_shape=jax.ShapeDtypeStruct((N, FO), jnp.float32),
    )(parts, cnt2, x, w3s, d3row,
      r(p['m2_g1']), r(p['m2_b1']), p['m2_W1'], r(p['m2_c1']),
      r(p['m2_g2']), r(p['m2_b2']), p['m2_W2'], r(p['m2_c2']),
      r(p['m2_g3']), r(p['m2_b3']), p['m2_W3'], r(p['m2_c3']))


def kernel(x, edge_index, edge_attr, u, batch, params):
    p = params
    row = edge_index[0]
    col = edge_index[1]

    z128 = jnp.zeros((NPAD, FO), jnp.float32)
    ones128 = jnp.ones((CH, FO), jnp.float32)

    cnt2 = _sc_counts(row, col, z128, ones128)
    easum = _tc_ea_stats(edge_attr)

    y, w1es, b1row = _tc_prep(
        x, cnt2, easum, p['m1_W1'],
        p['m1_g1'].reshape(1, D1), p['m1_b1'].reshape(1, D1),
        p['m1_c1'].reshape(1, FO))

    G = _sc_gather(y, row)

    st2 = _tc_pass_a(G, edge_attr, w1es, b1row)
    mu2 = st2[0] / E
    var2 = st2[1] / E - mu2 * mu2
    sc2 = p['m1_g2'] * lax.rsqrt(var2 + EPS)
    sh2 = p['m1_b2'] - mu2 * sc2
    w2s = p['m1_W2'] * sc2[None, :]
    b2row = (sh2 @ p['m1_W2'].T + p['m1_c2']).reshape(1, FO)

    e2, st3 = _tc_pass_b(G, edge_attr, w1es, b1row, w2s, b2row)
    mu3 = st3[0] / E
    var3 = st3[1] / E - mu3 * mu3
    sc3 = p['m1_g3'] * lax.rsqrt(var3 + EPS)
    sh3 = p['m1_b3'] - mu3 * sc3
    w3s = p['m1_W3'] * sc3[None, :]
    d3row = (sh3 @ p['m1_W3'].T + p['m1_c3']).reshape(1, FO)

    parts = _sc_scatter(e2, col, z128)

    return _tc_final(parts, cnt2, x, w3s, d3row, p)


# speedup vs baseline: 3.2207x; 1.1472x over previous
import functools

import jax
import jax.numpy as jnp
from jax import lax
from jax.experimental import pallas as pl
from jax.experimental.pallas import tpu as pltpu
from jax.experimental.pallas import tpu_sc as plsc

N = 10000
NPAD = 10240
E = 320000
FX = 128
FE = 16
FO = 128
D1 = FX + FE
D2 = FX + FO
EPS = 1e-5

NC = 2
NS = 16
NW = NC * NS
CH = 128
NCHUNK = E // CH
ROWS_PER_TILE = NPAD // NS

ECHUNK = 2560
EGRID = E // ECHUNK


def _mm_nt(a, b):
    return lax.dot_general(a, b, (((1,), (1,)), ((), ())),
                           preferred_element_type=jnp.float32)


def _elu(a):
    return jnp.where(a > 0, a, jnp.exp(a) - 1.0)


def _sc_mesh():
    return plsc.VectorSubcoreMesh(core_axis_name="c", subcore_axis_name="s",
                                  num_cores=NC, num_subcores=NS)


def _wid():
    return lax.axis_index("s") * NC + lax.axis_index("c")


def _sc_counts(row, col, z128, ones128):
    @functools.partial(
        pl.kernel,
        out_type=jax.ShapeDtypeStruct((NC, NPAD, FO), jnp.float32),
        mesh=_sc_mesh(),
        scratch_types=[pltpu.VMEM((2, CH), jnp.int32),
                       pltpu.VMEM((CH, FO), jnp.float32),
                       pltpu.VMEM_SHARED((NPAD, FO), jnp.float32),
                       pltpu.SemaphoreType.DMA,
                       pltpu.SemaphoreType.DMA],
    )
    def k(row_h, col_h, z_h, ones_h, out_h, idx_v, ones_v, acc, sa, sb):
        cid = lax.axis_index("c")
        sid = lax.axis_index("s")
        r0 = sid * ROWS_PER_TILE
        pltpu.sync_copy(ones_h, ones_v)
        pltpu.sync_copy(z_h.at[pl.ds(r0, ROWS_PER_TILE)],
                        acc.at[pl.ds(r0, ROWS_PER_TILE)])
        plsc.subcore_barrier()

        trips = (NCHUNK - sid + NS - 1) // NS
        pairs = trips // 2

        def load_idx(base, slot):
            @pl.when(cid == 0)
            def _():
                pltpu.sync_copy(row_h.at[pl.ds(base, CH)], idx_v.at[slot])

            @pl.when(cid == 1)
            def _():
                pltpu.sync_copy(col_h.at[pl.ds(base, CH)], idx_v.at[slot])

        def pair(i, carry):
            ba = (sid + (2 * i) * NS) * CH
            bb = (sid + (2 * i + 1) * NS) * CH
            load_idx(ba, 0)
            da = pltpu.async_copy(ones_v, acc.at[idx_v.at[0]], sa, add=True)
            load_idx(bb, 1)
            db = pltpu.async_copy(ones_v, acc.at[idx_v.at[1]], sb, add=True)
            da.wait()
            db.wait()
            return carry

        lax.fori_loop(0, pairs, pair, 0)

        @pl.when(trips % 2 == 1)
        def _():
            base = (sid + (trips - 1) * NS) * CH
            load_idx(base, 0)
            pltpu.sync_copy(ones_v, acc.at[idx_v.at[0]], add=True)

        plsc.subcore_barrier()
        pltpu.sync_copy(acc.at[pl.ds(r0, ROWS_PER_TILE)],
                        out_h.at[cid, pl.ds(r0, ROWS_PER_TILE)])

    return k(row, col, z128, ones128)


def _sc_gather(ytab, row):
    @functools.partial(
        pl.kernel,
        out_type=jax.ShapeDtypeStruct((E, FO), jnp.float32),
        mesh=_sc_mesh(),
        scratch_types=[pltpu.VMEM((2, CH), jnp.int32),
                       pltpu.VMEM((CH, FO), jnp.float32),
                       pltpu.VMEM((CH, FO), jnp.float32),
                       pltpu.SemaphoreType.DMA,
                       pltpu.SemaphoreType.DMA,
                       pltpu.SemaphoreType.DMA,
                       pltpu.SemaphoreType.DMA],
    )
    def k(y_h, row_h, g_h, idx_v, rows_a, rows_b, ga, gb, sa, sb):
        w = _wid()
        trips = (NCHUNK - w + NW - 1) // NW
        pairs = trips // 2

        def pair(i, carry):
            ba = (w + (2 * i) * NW) * CH
            bb = (w + (2 * i + 1) * NW) * CH
            pltpu.sync_copy(row_h.at[pl.ds(ba, CH)], idx_v.at[0])
            da = pltpu.async_copy(y_h.at[idx_v.at[0]], rows_a, ga)
            pltpu.sync_copy(row_h.at[pl.ds(bb, CH)], idx_v.at[1])
            db = pltpu.async_copy(y_h.at[idx_v.at[1]], rows_b, gb)
            da.wait()
            wa = pltpu.async_copy(rows_a, g_h.at[pl.ds(ba, CH)], sa)
            db.wait()
            wb = pltpu.async_copy(rows_b, g_h.at[pl.ds(bb, CH)], sb)
            wa.wait()
            wb.wait()
            return carry

        lax.fori_loop(0, pairs, pair, 0)

        @pl.when(trips % 2 == 1)
        def _():
            base = (w + (trips - 1) * NW) * CH
            pltpu.sync_copy(row_h.at[pl.ds(base, CH)], idx_v.at[0])
            pltpu.async_copy(y_h.at[idx_v.at[0]], rows_a, ga).wait()
            pltpu.sync_copy(rows_a, g_h.at[pl.ds(base, CH)])

    return k(ytab, row)


def _sc_scatter(e2, col, z128):
    @functools.partial(
        pl.kernel,
        out_type=jax.ShapeDtypeStruct((NC, NPAD, FO), jnp.float32),
        mesh=_sc_mesh(),
        scratch_types=[pltpu.VMEM((2, CH), jnp.int32),
                       pltpu.VMEM((CH, FO), jnp.float32),
                       pltpu.VMEM((CH, FO), jnp.float32),
                       pltpu.VMEM_SHARED((NPAD, FO), jnp.float32),
                       pltpu.SemaphoreType.DMA,
                       pltpu.SemaphoreType.DMA,
                       pltpu.SemaphoreType.DMA,
                       pltpu.SemaphoreType.DMA],
    )
    def k(e2_h, col_h, z_h, out_h, idx_v, val_a, val_b, acc, la, lb, sa, sb):
        cid = lax.axis_index("c")
        sid = lax.axis_index("s")
        w = _wid()
        r0 = sid * ROWS_PER_TILE
        pltpu.sync_copy(z_h.at[pl.ds(r0, ROWS_PER_TILE)],
                        acc.at[pl.ds(r0, ROWS_PER_TILE)])
        plsc.subcore_barrier()

        trips = (NCHUNK - w + NW - 1) // NW
        pairs = trips // 2

        def pair(i, carry):
            ba = (w + (2 * i) * NW) * CH
            bb = (w + (2 * i + 1) * NW) * CH
            pltpu.sync_copy(col_h.at[pl.ds(ba, CH)], idx_v.at[0])
            da = pltpu.async_copy(e2_h.at[pl.ds(ba, CH)], val_a, la)
            pltpu.sync_copy(col_h.at[pl.ds(bb, CH)], idx_v.at[1])
            db = pltpu.async_copy(e2_h.at[pl.ds(bb, CH)], val_b, lb)
            da.wait()
            wa = pltpu.async_copy(val_a, acc.at[idx_v.at[0]], sa, add=True)
            db.wait()
            wb = pltpu.async_copy(val_b, acc.at[idx_v.at[1]], sb, add=True)
            wa.wait()
            wb.wait()
            return carry

        lax.fori_loop(0, pairs, pair, 0)

        @pl.when(trips % 2 == 1)
        def _():
            base = (w + (trips - 1) * NW) * CH
            pltpu.sync_copy(col_h.at[pl.ds(base, CH)], idx_v.at[0])
            pltpu.sync_copy(e2_h.at[pl.ds(base, CH)], val_a)
            pltpu.sync_copy(val_a, acc.at[idx_v.at[0]], add=True)

        plsc.subcore_barrier()
        pltpu.sync_copy(acc.at[pl.ds(r0, ROWS_PER_TILE)],
                        out_h.at[cid, pl.ds(r0, ROWS_PER_TILE)])

    return k(e2, col, z128)


def _tc_ea_stats(ea):
    def body(ea_ref, out_ref, acc_ref):
        i = pl.program_id(0)

        @pl.when(i == 0)
        def _():
            acc_ref[...] = jnp.zeros_like(acc_ref)

        blk = ea_ref[...]
        acc_ref[0:1, :] += jnp.sum(blk, axis=0, keepdims=True)
        acc_ref[1:2, :] += jnp.sum(blk * blk, axis=0, keepdims=True)

        @pl.when(i == EGRID - 1)
        def _():
            out_ref[...] = acc_ref[...]

    return pl.pallas_call(
        body,
        grid=(EGRID,),
        in_specs=[pl.BlockSpec((ECHUNK, FE), lambda i: (i, 0))],
        out_specs=pl.BlockSpec((8, FE), lambda i: (0, 0)),
        out_shape=jax.ShapeDtypeStruct((8, FE), jnp.float32),
        scratch_shapes=[pltpu.VMEM((8, FE), jnp.float32)],
    )(ea)


def _tc_prep(x, cnt2, easum, W1, g1r, b1r, c1r):
    def body(x_ref, cnt_ref, eas_ref, w1_ref, g1_ref, b1_ref, c1_ref,
             y_ref, w1es_ref, b1row_ref):
        cnt = cnt_ref[0, :N, 0:1]
        xv = x_ref[...]
        einv = 1.0 / E
        sum_x = jnp.sum(xv * cnt, axis=0, keepdims=True)
        sumsq_x = jnp.sum(xv * xv * cnt, axis=0, keepdims=True)
        mu_x = sum_x * einv
        var_x = sumsq_x * einv - mu_x * mu_x
        sc_x = g1_ref[0:1, :FX] * lax.rsqrt(var_x + EPS)
        sh_x = b1_ref[0:1, :FX] - mu_x * sc_x

        mu_e = eas_ref[0:1, :] * einv
        var_e = eas_ref[1:2, :] * einv - mu_e * mu_e
        sc_e = g1_ref[0:1, FX:] * lax.rsqrt(var_e + EPS)
        sh_e = b1_ref[0:1, FX:] - mu_e * sc_e

        w1 = w1_ref[...]
        w1x = w1[:, :FX]
        w1e = w1[:, FX:]
        y_ref[...] = _mm_nt(xv, w1x * sc_x)
        w1es_ref[...] = w1e * sc_e
        b1row_ref[...] = _mm_nt(sh_x, w1x) + _mm_nt(sh_e, w1e) + c1_ref[...]

    return pl.pallas_call(
        body,
        out_shape=[jax.ShapeDtypeStruct((N, FO), jnp.float32),
                   jax.ShapeDtypeStruct((FO, FE), jnp.float32),
                   jax.ShapeDtypeStruct((1, FO), jnp.float32)],
    )(x, cnt2, easum, W1, g1r, b1r, c1r)


def _tc_pass_a(G, ea, w1es, b1row):
    def body(g_ref, ea_ref, w_ref, b_ref, out_ref, acc_ref):
        i = pl.program_id(0)

        @pl.when(i == 0)
        def _():
            acc_ref[...] = jnp.zeros_like(acc_ref)

        a = g_ref[...] + _mm_nt(ea_ref[...], w_ref[...]) + b_ref[...]
        e1 = _elu(a)
        acc_ref[0:1, :] += jnp.sum(e1, axis=0, keepdims=True)
        acc_ref[1:2, :] += jnp.sum(e1 * e1, axis=0, keepdims=True)

        @pl.when(i == EGRID - 1)
        def _():
            out_ref[...] = acc_ref[...]

    return pl.pallas_call(
        body,
        grid=(EGRID,),
        in_specs=[pl.BlockSpec((ECHUNK, FO), lambda i: (i, 0)),
                  pl.BlockSpec((ECHUNK, FE), lambda i: (i, 0)),
                  pl.BlockSpec((FO, FE), lambda i: (0, 0)),
                  pl.BlockSpec((1, FO), lambda i: (0, 0))],
        out_specs=pl.BlockSpec((8, FO), lambda i: (0, 0)),
        out_shape=jax.ShapeDtypeStruct((8, FO), jnp.float32),
        scratch_shapes=[pltpu.VMEM((8, FO), jnp.float32)],
    )(G, ea, w1es, b1row)


def _tc_pass_b(G, ea, w1es, b1row, w2s, b2row):
    def body(g_ref, ea_ref, w_ref, b_ref, w2_ref, b2_ref,
             e2_ref, out_ref, acc_ref):
        i = pl.program_id(0)

        @pl.when(i == 0)
        def _():
            acc_ref[...] = jnp.zeros_like(acc_ref)

        a = g_ref[...] + _mm_nt(ea_ref[...], w_ref[...]) + b_ref[...]
        e1 = _elu(a)
        e2 = _elu(_mm_nt(e1, w2_ref[...]) + b2_ref[...])
        e2_ref[...] = e2
        acc_ref[0:1, :] += jnp.sum(e2, axis=0, keepdims=True)
        acc_ref[1:2, :] += jnp.sum(e2 * e2, axis=0, keepdims=True)

        @pl.when(i == EGRID - 1)
        def _():
            out_ref[...] = acc_ref[...]

    return pl.pallas_call(
        body,
        grid=(EGRID,),
        in_specs=[pl.BlockSpec((ECHUNK, FO), lambda i: (i, 0)),
                  pl.BlockSpec((ECHUNK, FE), lambda i: (i, 0)),
                  pl.BlockSpec((FO, FE), lambda i: (0, 0)),
                  pl.BlockSpec((1, FO), lambda i: (0, 0)),
                  pl.BlockSpec((FO, FO), lambda i: (0, 0)),
                  pl.BlockSpec((1, FO), lambda i: (0, 0))],
        out_specs=[pl.BlockSpec((ECHUNK, FO), lambda i: (i, 0)),
                   pl.BlockSpec((8, FO), lambda i: (0, 0))],
        out_shape=[jax.ShapeDtypeStruct((E, FO), jnp.float32),
                   jax.ShapeDtypeStruct((8, FO), jnp.float32)],
        scratch_shapes=[pltpu.VMEM((8, FO), jnp.float32)],
    )(G, ea, w1es, b1row, w2s, b2row)


def _tc_final(parts, cnt2, x, w3s, d3row, p):
    def body(parts_ref, cnt_ref, x_ref, w3_ref, d3_ref,
             g1_ref, b1_ref, mw1_ref, c1_ref,
             g2_ref, b2_ref, mw2_ref, c2_ref,
             g3_ref, b3_ref, mw3_ref, c3_ref, out_ref):
        seg = parts_ref[0, :N, :] + parts_ref[1, :N, :]
        cnt = cnt_ref[1, :N, 0:1]
        sums = _mm_nt(seg, w3_ref[...]) + cnt * d3_ref[...]
        mean = sums / jnp.maximum(cnt, 1.0)
        h = jnp.concatenate([x_ref[...], mean], axis=1)

        def bn(v, g, b):
            mu = jnp.mean(v, axis=0, keepdims=True)
            var = jnp.mean(v * v, axis=0, keepdims=True) - mu * mu
            s = g * lax.rsqrt(var + EPS)
            return v * s + (b - mu * s)

        h = bn(h, g1_ref[...], b1_ref[...])
        h = _elu(_mm_nt(h, mw1_ref[...]) + c1_ref[...])
        h = bn(h, g2_ref[...], b2_ref[...])
        h = _elu(_mm_nt(h, mw2_ref[...]) + c2_ref[...])
        h = bn(h, g3_ref[...], b3_ref[...])
        out_ref[...] = _mm_nt(h, mw3_ref[...]) + c3_ref[...]

    r = lambda v: v.reshape(1, -1)
    return pl.pallas_call(
        body,
        out_shape=jax.ShapeDtypeStruct((N, FO), jnp.float32),
    )(parts, cnt2, x, w3s, d3row,
      r(p['m2_g1']), r(p['m2_b1']), p['m2_W1'], r(p['m2_c1']),
      r(p['m2_g2']), r(p['m2_b2']), p['m2_W2'], r(p['m2_c2']),
      r(p['m2_g3']), r(p['m2_b3']), p['m2_W3'], r(p['m2_c3']))


def kernel(x, edge_index, edge_attr, u, batch, params):
    p = params
    row = edge_index[0]
    col = edge_index[1]

    z128 = jnp.zeros((NPAD, FO), jnp.float32)
    ones128 = jnp.ones((CH, FO), jnp.float32)

    cnt2 = _sc_counts(row, col, z128, ones128)
    easum = _tc_ea_stats(edge_attr)

    y, w1es, b1row = _tc_prep(
        x, cnt2, easum, p['m1_W1'],
        p['m1_g1'].reshape(1, D1), p['m1_b1'].reshape(1, D1),
        p['m1_c1'].reshape(1, FO))

    G = _sc_gather(y, row)

    st2 = _tc_pass_a(G, edge_attr, w1es, b1row)
    mu2 = st2[0] / E
    var2 = st2[1] / E - mu2 * mu2
    sc2 = p['m1_g2'] * lax.rsqrt(var2 + EPS)
    sh2 = p['m1_b2'] - mu2 * sc2
    w2s = p['m1_W2'] * sc2[None, :]
    b2row = (sh2 @ p['m1_W2'].T + p['m1_c2']).reshape(1, FO)

    e2, st3 = _tc_pass_b(G, edge_attr, w1es, b1row, w2s, b2row)
    mu3 = st3[0] / E
    var3 = st3[1] / E - mu3 * mu3
    sc3 = p['m1_g3'] * lax.rsqrt(var3 + EPS)
    sh3 = p['m1_b3'] - mu3 * sc3
    w3s = p['m1_W3'] * sc3[None, :]
    d3row = (sh3 @ p['m1_W3'].T + p['m1_c3']).reshape(1, FO)

    parts = _sc_scatter(e2, col, z128)

    return _tc_final(parts, cnt2, x, w3s, d3row, p)


# speedup vs baseline: 3.2927x; 1.0224x over previous
import functools

import jax
import jax.numpy as jnp
from jax import lax
from jax.experimental import pallas as pl
from jax.experimental.pallas import tpu as pltpu
from jax.experimental.pallas import tpu_sc as plsc

N = 10000
NPAD = 10240
E = 320000
FX = 128
FE = 16
FO = 128
D1 = FX + FE
D2 = FX + FO
EPS = 1e-5

NC = 2
NS = 16
NW = NC * NS
CH = 128
NCHUNK = E // CH
ROWS_PER_TILE = NPAD // NS

ECHUNK = 2560
EGRID = E // ECHUNK


def _mm_nt(a, b):
    return lax.dot_general(a, b, (((1,), (1,)), ((), ())),
                           preferred_element_type=jnp.float32)


def _elu(a):
    return jnp.where(a > 0, a, jnp.exp(a) - 1.0)


def _sc_mesh():
    return plsc.VectorSubcoreMesh(core_axis_name="c", subcore_axis_name="s",
                                  num_cores=NC, num_subcores=NS)


def _wid():
    return lax.axis_index("s") * NC + lax.axis_index("c")


def _sc_counts(row, col, z128, ones128):
    @functools.partial(
        pl.kernel,
        out_type=jax.ShapeDtypeStruct((NC, NPAD, FO), jnp.float32),
        mesh=_sc_mesh(),
        scratch_types=[pltpu.VMEM((4, CH), jnp.int32),
                       pltpu.VMEM((CH, FO), jnp.float32),
                       pltpu.VMEM_SHARED((NPAD, FO), jnp.float32),
                       pltpu.SemaphoreType.DMA,
                       pltpu.SemaphoreType.DMA,
                       pltpu.SemaphoreType.DMA,
                       pltpu.SemaphoreType.DMA],
    )
    def k(row_h, col_h, z_h, ones_h, out_h, idx_v, ones_v, acc, s0, s1, s2, s3):
        cid = lax.axis_index("c")
        sid = lax.axis_index("s")
        ssem = [s0, s1, s2, s3]
        r0 = sid * ROWS_PER_TILE
        pltpu.sync_copy(ones_h, ones_v)
        pltpu.sync_copy(z_h.at[pl.ds(r0, ROWS_PER_TILE)],
                        acc.at[pl.ds(r0, ROWS_PER_TILE)])
        plsc.subcore_barrier()

        trips = (NCHUNK - sid + NS - 1) // NS
        quads = trips // 4

        def load_idx(base, slot):
            @pl.when(cid == 0)
            def _():
                pltpu.sync_copy(row_h.at[pl.ds(base, CH)], idx_v.at[slot])

            @pl.when(cid == 1)
            def _():
                pltpu.sync_copy(col_h.at[pl.ds(base, CH)], idx_v.at[slot])

        def quad(i, carry):
            descs = []
            for b in range(4):
                base = (sid + (4 * i + b) * NS) * CH
                load_idx(base, b)
                descs.append(pltpu.async_copy(ones_v, acc.at[idx_v.at[b]],
                                              ssem[b], add=True))
            for b in range(4):
                descs[b].wait()
            return carry

        lax.fori_loop(0, quads, quad, 0)

        def tail(j, carry):
            base = (sid + (quads * 4 + j) * NS) * CH
            load_idx(base, 0)
            pltpu.sync_copy(ones_v, acc.at[idx_v.at[0]], add=True)
            return carry

        lax.fori_loop(0, trips - quads * 4, tail, 0)
        plsc.subcore_barrier()
        pltpu.sync_copy(acc.at[pl.ds(r0, ROWS_PER_TILE)],
                        out_h.at[cid, pl.ds(r0, ROWS_PER_TILE)])

    return k(row, col, z128, ones128)


def _sc_gather(ytab, row):
    @functools.partial(
        pl.kernel,
        out_type=jax.ShapeDtypeStruct((E, FO), jnp.float32),
        mesh=_sc_mesh(),
        scratch_types=[pltpu.VMEM((4, CH), jnp.int32),
                       pltpu.VMEM((CH, FO), jnp.float32),
                       pltpu.VMEM((CH, FO), jnp.float32),
                       pltpu.VMEM((CH, FO), jnp.float32),
                       pltpu.VMEM((CH, FO), jnp.float32),
                       pltpu.SemaphoreType.DMA,
                       pltpu.SemaphoreType.DMA,
                       pltpu.SemaphoreType.DMA,
                       pltpu.SemaphoreType.DMA,
                       pltpu.SemaphoreType.DMA,
                       pltpu.SemaphoreType.DMA,
                       pltpu.SemaphoreType.DMA,
                       pltpu.SemaphoreType.DMA],
    )
    def k(y_h, row_h, g_h, idx_v, rows_a, rows_b, rows_c, rows_d,
          g0, g1, g2, g3, s0, s1, s2, s3):
        w = _wid()
        rows = [rows_a, rows_b, rows_c, rows_d]
        gsem = [g0, g1, g2, g3]
        ssem = [s0, s1, s2, s3]
        trips = (NCHUNK - w + NW - 1) // NW
        quads = trips // 4

        def quad(i, carry):
            bases = [(w + (4 * i + b) * NW) * CH for b in range(4)]
            descs = []
            for b in range(4):
                pltpu.sync_copy(row_h.at[pl.ds(bases[b], CH)], idx_v.at[b])
                descs.append(pltpu.async_copy(y_h.at[idx_v.at[b]], rows[b], gsem[b]))
            sts = []
            for b in range(4):
                descs[b].wait()
                sts.append(pltpu.async_copy(rows[b], g_h.at[pl.ds(bases[b], CH)], ssem[b]))
            for b in range(4):
                sts[b].wait()
            return carry

        lax.fori_loop(0, quads, quad, 0)

        def tail(j, carry):
            base = (w + (quads * 4 + j) * NW) * CH
            pltpu.sync_copy(row_h.at[pl.ds(base, CH)], idx_v.at[0])
            pltpu.async_copy(y_h.at[idx_v.at[0]], rows_a, g0).wait()
            pltpu.sync_copy(rows_a, g_h.at[pl.ds(base, CH)])
            return carry

        lax.fori_loop(0, trips - quads * 4, tail, 0)

    return k(ytab, row)


def _sc_scatter(e2, col, z128):
    @functools.partial(
        pl.kernel,
        out_type=jax.ShapeDtypeStruct((NC, NPAD, FO), jnp.float32),
        mesh=_sc_mesh(),
        scratch_types=[pltpu.VMEM((2, CH), jnp.int32),
                       pltpu.VMEM((CH, FO), jnp.float32),
                       pltpu.VMEM((CH, FO), jnp.float32),
                       pltpu.VMEM_SHARED((NPAD, FO), jnp.float32),
                       pltpu.SemaphoreType.DMA,
                       pltpu.SemaphoreType.DMA,
                       pltpu.SemaphoreType.DMA,
                       pltpu.SemaphoreType.DMA],
    )
    def k(e2_h, col_h, z_h, out_h, idx_v, val_a, val_b, acc, la, lb, sa, sb):
        cid = lax.axis_index("c")
        sid = lax.axis_index("s")
        w = _wid()
        r0 = sid * ROWS_PER_TILE
        pltpu.sync_copy(z_h.at[pl.ds(r0, ROWS_PER_TILE)],
                        acc.at[pl.ds(r0, ROWS_PER_TILE)])
        plsc.subcore_barrier()

        trips = (NCHUNK - w + NW - 1) // NW
        pairs = trips // 2

        def pair(i, carry):
            ba = (w + (2 * i) * NW) * CH
            bb = (w + (2 * i + 1) * NW) * CH
            pltpu.sync_copy(col_h.at[pl.ds(ba, CH)], idx_v.at[0])
            da = pltpu.async_copy(e2_h.at[pl.ds(ba, CH)], val_a, la)
            pltpu.sync_copy(col_h.at[pl.ds(bb, CH)], idx_v.at[1])
            db = pltpu.async_copy(e2_h.at[pl.ds(bb, CH)], val_b, lb)
            da.wait()
            wa = pltpu.async_copy(val_a, acc.at[idx_v.at[0]], sa, add=True)
            db.wait()
            wb = pltpu.async_copy(val_b, acc.at[idx_v.at[1]], sb, add=True)
            wa.wait()
            wb.wait()
            return carry

        lax.fori_loop(0, pairs, pair, 0)

        @pl.when(trips % 2 == 1)
        def _():
            base = (w + (trips - 1) * NW) * CH
            pltpu.sync_copy(col_h.at[pl.ds(base, CH)], idx_v.at[0])
            pltpu.sync_copy(e2_h.at[pl.ds(base, CH)], val_a)
            pltpu.sync_copy(val_a, acc.at[idx_v.at[0]], add=True)

        plsc.subcore_barrier()
        pltpu.sync_copy(acc.at[pl.ds(r0, ROWS_PER_TILE)],
                        out_h.at[cid, pl.ds(r0, ROWS_PER_TILE)])

    return k(e2, col, z128)


def _tc_ea_stats(ea):
    def body(ea_ref, out_ref, acc_ref):
        i = pl.program_id(0)

        @pl.when(i == 0)
        def _():
            acc_ref[...] = jnp.zeros_like(acc_ref)

        blk = ea_ref[...]
        acc_ref[0:1, :] += jnp.sum(blk, axis=0, keepdims=True)
        acc_ref[1:2, :] += jnp.sum(blk * blk, axis=0, keepdims=True)

        @pl.when(i == EGRID - 1)
        def _():
            out_ref[...] = acc_ref[...]

    return pl.pallas_call(
        body,
        grid=(EGRID,),
        in_specs=[pl.BlockSpec((ECHUNK, FE), lambda i: (i, 0))],
        out_specs=pl.BlockSpec((8, FE), lambda i: (0, 0)),
        out_shape=jax.ShapeDtypeStruct((8, FE), jnp.float32),
        scratch_shapes=[pltpu.VMEM((8, FE), jnp.float32)],
    )(ea)


def _tc_prep(x, cnt2, easum, W1, g1r, b1r, c1r):
    def body(x_ref, cnt_ref, eas_ref, w1_ref, g1_ref, b1_ref, c1_ref,
             y_ref, w1es_ref, b1row_ref):
        cnt = cnt_ref[0, :N, 0:1]
        xv = x_ref[...]
        einv = 1.0 / E
        sum_x = jnp.sum(xv * cnt, axis=0, keepdims=True)
        sumsq_x = jnp.sum(xv * xv * cnt, axis=0, keepdims=True)
        mu_x = sum_x * einv
        var_x = sumsq_x * einv - mu_x * mu_x
        sc_x = g1_ref[0:1, :FX] * lax.rsqrt(var_x + EPS)
        sh_x = b1_ref[0:1, :FX] - mu_x * sc_x

        mu_e = eas_ref[0:1, :] * einv
        var_e = eas_ref[1:2, :] * einv - mu_e * mu_e
        sc_e = g1_ref[0:1, FX:] * lax.rsqrt(var_e + EPS)
        sh_e = b1_ref[0:1, FX:] - mu_e * sc_e

        w1 = w1_ref[...]
        w1x = w1[:, :FX]
        w1e = w1[:, FX:]
        y_ref[...] = _mm_nt(xv, w1x * sc_x)
        w1es_ref[...] = w1e * sc_e
        b1row_ref[...] = _mm_nt(sh_x, w1x) + _mm_nt(sh_e, w1e) + c1_ref[...]

    return pl.pallas_call(
        body,
        out_shape=[jax.ShapeDtypeStruct((N, FO), jnp.float32),
                   jax.ShapeDtypeStruct((FO, FE), jnp.float32),
                   jax.ShapeDtypeStruct((1, FO), jnp.float32)],
    )(x, cnt2, easum, W1, g1r, b1r, c1r)


def _tc_pass_a(G, ea, w1es, b1row):
    def body(g_ref, ea_ref, w_ref, b_ref, out_ref, acc_ref):
        i = pl.program_id(0)

        @pl.when(i == 0)
        def _():
            acc_ref[...] = jnp.zeros_like(acc_ref)

        a = g_ref[...] + _mm_nt(ea_ref[...], w_ref[...]) + b_ref[...]
        e1 = _elu(a)
        acc_ref[0:1, :] += jnp.sum(e1, axis=0, keepdims=True)
        acc_ref[1:2, :] += jnp.sum(e1 * e1, axis=0, keepdims=True)

        @pl.when(i == EGRID - 1)
        def _():
            out_ref[...] = acc_ref[...]

    return pl.pallas_call(
        body,
        grid=(EGRID,),
        in_specs=[pl.BlockSpec((ECHUNK, FO), lambda i: (i, 0)),
                  pl.BlockSpec((ECHUNK, FE), lambda i: (i, 0)),
                  pl.BlockSpec((FO, FE), lambda i: (0, 0)),
                  pl.BlockSpec((1, FO), lambda i: (0, 0))],
        out_specs=pl.BlockSpec((8, FO), lambda i: (0, 0)),
        out_shape=jax.ShapeDtypeStruct((8, FO), jnp.float32),
        scratch_shapes=[pltpu.VMEM((8, FO), jnp.float32)],
    )(G, ea, w1es, b1row)


def _tc_pass_b(G, ea, w1es, b1row, w2s, b2row):
    def body(g_ref, ea_ref, w_ref, b_ref, w2_ref, b2_ref,
             e2_ref, out_ref, acc_ref):
        i = pl.program_id(0)

        @pl.when(i == 0)
        def _():
            acc_ref[...] = jnp.zeros_like(acc_ref)

        a = g_ref[...] + _mm_nt(ea_ref[...], w_ref[...]) + b_ref[...]
        e1 = _elu(a)
        e2 = _elu(_mm_nt(e1, w2_ref[...]) + b2_ref[...])
        e2_ref[...] = e2
        acc_ref[0:1, :] += jnp.sum(e2, axis=0, keepdims=True)
        acc_ref[1:2, :] += jnp.sum(e2 * e2, axis=0, keepdims=True)

        @pl.when(i == EGRID - 1)
        def _():
            out_ref[...] = acc_ref[...]

    return pl.pallas_call(
        body,
        grid=(EGRID,),
        in_specs=[pl.BlockSpec((ECHUNK, FO), lambda i: (i, 0)),
                  pl.BlockSpec((ECHUNK, FE), lambda i: (i, 0)),
                  pl.BlockSpec((FO, FE), lambda i: (0, 0)),
                  pl.BlockSpec((1, FO), lambda i: (0, 0)),
                  pl.BlockSpec((FO, FO), lambda i: (0, 0)),
                  pl.BlockSpec((1, FO), lambda i: (0, 0))],
        out_specs=[pl.BlockSpec((ECHUNK, FO), lambda i: (i, 0)),
                   pl.BlockSpec((8, FO), lambda i: (0, 0))],
        out_shape=[jax.ShapeDtypeStruct((E, FO), jnp.float32),
                   jax.ShapeDtypeStruct((8, FO), jnp.float32)],
        scratch_shapes=[pltpu.VMEM((8, FO), jnp.float32)],
    )(G, ea, w1es, b1row, w2s, b2row)


def _tc_final(parts, cnt2, x, w3s, d3row, p):
    def body(parts_ref, cnt_ref, x_ref, w3_ref, d3_ref,
             g1_ref, b1_ref, mw1_ref, c1_ref,
             g2_ref, b2_ref, mw2_ref, c2_ref,
             g3_ref, b3_ref, mw3_ref, c3_ref, out_ref):
        seg = parts_ref[0, :N, :] + parts_ref[1, :N, :]
        cnt = cnt_ref[1, :N, 0:1]
        sums = _mm_nt(seg, w3_ref[...]) + cnt * d3_ref[...]
        mean = sums / jnp.maximum(cnt, 1.0)
        h = jnp.concatenate([x_ref[...], mean], axis=1)

        def bn(v, g, b):
            mu = jnp.mean(v, axis=0, keepdims=True)
            var = jnp.mean(v * v, axis=0, keepdims=True) - mu * mu
            s = g * lax.rsqrt(var + EPS)
            return v * s + (b - mu * s)

        h = bn(h, g1_ref[...], b1_ref[...])
        h = _elu(_mm_nt(h, mw1_ref[...]) + c1_ref[...])
        h = bn(h, g2_ref[...], b2_ref[...])
        h = _elu(_mm_nt(h, mw2_ref[...]) + c2_ref[...])
        h = bn(h, g3_ref[...], b3_ref[...])
        out_ref[...] = _mm_nt(h, mw3_ref[...]) + c3_ref[...]

    r = lambda v: v.reshape(1, -1)
    return pl.pallas_call(
        body,
        out_shape=jax.ShapeDtypeStruct((N, FO), jnp.float32),
    )(parts, cnt2, x, w3s, d3row,
      r(p['m2_g1']), r(p['m2_b1']), p['m2_W1'], r(p['m2_c1']),
      r(p['m2_g2']), r(p['m2_b2']), p['m2_W2'], r(p['m2_c2']),
      r(p['m2_g3']), r(p['m2_b3']), p['m2_W3'], r(p['m2_c3']))


def kernel(x, edge_index, edge_attr, u, batch, params):
    p = params
    row = edge_index[0]
    col = edge_index[1]

    z128 = jnp.zeros((NPAD, FO), jnp.float32)
    ones128 = jnp.ones((CH, FO), jnp.float32)

    cnt2 = _sc_counts(row, col, z128, ones128)
    easum = _tc_ea_stats(edge_attr)

    y, w1es, b1row = _tc_prep(
        x, cnt2, easum, p['m1_W1'],
        p['m1_g1'].reshape(1, D1), p['m1_b1'].reshape(1, D1),
        p['m1_c1'].reshape(1, FO))

    G = _sc_gather(y, row)

    st2 = _tc_pass_a(G, edge_attr, w1es, b1row)
    mu2 = st2[0] / E
    var2 = st2[1] / E - mu2 * mu2
    sc2 = p['m1_g2'] * lax.rsqrt(var2 + EPS)
    sh2 = p['m1_b2'] - mu2 * sc2
    w2s = p['m1_W2'] * sc2[None, :]
    b2row = (sh2 @ p['m1_W2'].T + p['m1_c2']).reshape(1, FO)

    e2, st3 = _tc_pass_b(G, edge_attr, w1es, b1row, w2s, b2row)
    mu3 = st3[0] / E
    var3 = st3[1] / E - mu3 * mu3
    sc3 = p['m1_g3'] * lax.rsqrt(var3 + EPS)
    sh3 = p['m1_b3'] - mu3 * sc3
    w3s = p['m1_W3'] * sc3[None, :]
    d3row = (sh3 @ p['m1_W3'].T + p['m1_c3']).reshape(1, FO)

    parts = _sc_scatter(e2, col, z128)

    return _tc_final(parts, cnt2, x, w3s, d3row, p)
